# Initial kernel scaffold; baseline (speedup 1.0000x reference)
import jax
import jax.numpy as jnp
from jax.experimental import pallas as pl


def kernel(v_net_x, v_net_edge_index, v_net_batch, p_net_x, p_net_edge_index, p_net_batch, curr_v_node_id, v_Wl, v_bl, v_W1, v_b1, v_W2, v_b2, p_Wl, p_bl, p_W1, p_b1, p_W2, p_b2):
    raise NotImplementedError("write your pallas kernel here")


# speedup vs baseline: 8.3039x; 8.3039x over previous
import functools

import jax
import jax.numpy as jnp
from jax import lax
from jax.experimental import pallas as pl
from jax.experimental.pallas import tpu as pltpu
from jax.experimental.pallas import tpu_sc as plsc

N = 10000
D = 128
DH = 64
N_PAD = 10240
NC, NS, L = 2, 16, 16
NW = NC * NS
EB = 128
DEG_W = 16
R_BLK = 512
N_GRID = N_PAD // R_BLK

_mesh = plsc.VectorSubcoreMesh(
    core_axis_name="c", subcore_axis_name="s", num_cores=NC, num_subcores=NS)


def _pad_edges(edge_index, e_pad):
    src = edge_index[0]
    dst = edge_index[1]
    e = src.shape[0]
    pad = e_pad - e
    src = jnp.concatenate([src, jnp.zeros((pad,), jnp.int32)])
    dst = jnp.concatenate([dst, jnp.full((pad,), N, jnp.int32)])
    return (src.reshape(NW, -1, EB), dst.reshape(NW, -1, EB))


def _make_deg_kernel(nb_v, nb_p):
    @functools.partial(
        pl.kernel,
        out_type=(
            jax.ShapeDtypeStruct((NC, N_PAD, DEG_W), jnp.float32),
            jax.ShapeDtypeStruct((NC, N_PAD, DEG_W), jnp.float32),
        ),
        mesh=_mesh,
        compiler_params=pltpu.CompilerParams(use_tc_tiling_on_sc=False),
        scratch_types=[
            pltpu.VMEM((max(nb_v, nb_p), EB), jnp.int32),
            pltpu.VMEM((EB, DEG_W), jnp.float32),
            pltpu.VMEM_SHARED((N_PAD, DEG_W), jnp.float32),
            pltpu.VMEM_SHARED((N_PAD, DEG_W), jnp.float32),
        ],
    )
    def deg_kernel(dstv_hbm, dstp_hbm, ones_hbm, zeros_hbm,
                   outv_hbm, outp_hbm, didx, onesv, tabv, tabp):
        cid = lax.axis_index("c")
        sid = lax.axis_index("s")
        wid = sid * NC + cid
        rpt = N_PAD // NS
        pltpu.sync_copy(ones_hbm, onesv)
        pltpu.sync_copy(zeros_hbm.at[pl.ds(sid * rpt, rpt)],
                        tabv.at[pl.ds(sid * rpt, rpt)])
        pltpu.sync_copy(zeros_hbm.at[pl.ds(sid * rpt, rpt)],
                        tabp.at[pl.ds(sid * rpt, rpt)])
        plsc.subcore_barrier()

        def body_v(i, carry):
            pltpu.sync_copy(dstv_hbm.at[wid].at[i], didx.at[i])
            pltpu.sync_copy(onesv, tabv.at[didx.at[i]], add=True)
            return carry

        lax.fori_loop(0, nb_v, body_v, 0)

        def body_p(i, carry):
            pltpu.sync_copy(dstp_hbm.at[wid].at[i], didx.at[i])
            pltpu.sync_copy(onesv, tabp.at[didx.at[i]], add=True)
            return carry

        lax.fori_loop(0, nb_p, body_p, 0)
        plsc.subcore_barrier()
        pltpu.sync_copy(tabv.at[pl.ds(sid * rpt, rpt)],
                        outv_hbm.at[cid].at[pl.ds(sid * rpt, rpt)])
        pltpu.sync_copy(tabp.at[pl.ds(sid * rpt, rpt)],
                        outp_hbm.at[cid].at[pl.ds(sid * rpt, rpt)])

    return deg_kernel


def _make_scatter_kernel(nb):
    @functools.partial(
        pl.kernel,
        out_type=jax.ShapeDtypeStruct((2, NC, N_PAD, DH), jnp.float32),
        mesh=_mesh,
        compiler_params=pltpu.CompilerParams(use_tc_tiling_on_sc=False),
        scratch_types=[
            pltpu.VMEM((nb, EB), jnp.int32),
            pltpu.VMEM((nb, EB), jnp.int32),
            pltpu.VMEM((EB, DH), jnp.float32),
            pltpu.VMEM((EB, DH), jnp.float32),
            pltpu.VMEM_SHARED((N_PAD, DH), jnp.float32),
            pltpu.SemaphoreType.DMA,
            pltpu.SemaphoreType.DMA,
        ],
    )
    def scatter_kernel(ua_hbm, ub_hbm, src_hbm, dst_hbm, zeros_hbm, out_hbm,
                       sidx, didx, rows_a, rows_b, acc, sem_a, sem_b):
        cid = lax.axis_index("c")
        sid = lax.axis_index("s")
        wid = sid * NC + cid
        rpt = N_PAD // NS

        pltpu.sync_copy(src_hbm.at[wid], sidx)
        pltpu.sync_copy(dst_hbm.at[wid], didx)

        def phase(u_hbm, ph):
            pltpu.sync_copy(zeros_hbm.at[pl.ds(sid * rpt, rpt)],
                            acc.at[pl.ds(sid * rpt, rpt)])
            plsc.subcore_barrier()
            pltpu.async_copy(u_hbm.at[sidx.at[0]], rows_a, sem_a)

            def body(j, carry):
                i0 = 2 * j
                pltpu.async_copy(u_hbm.at[sidx.at[i0 + 1]], rows_b, sem_b)
                pltpu.make_async_copy(
                    u_hbm.at[sidx.at[i0]], rows_a, sem_a).wait()
                pltpu.sync_copy(rows_a, acc.at[didx.at[i0]], add=True)

                @pl.when(j < nb // 2 - 1)
                def _():
                    pltpu.async_copy(u_hbm.at[sidx.at[i0 + 2]], rows_a, sem_a)

                pltpu.make_async_copy(
                    u_hbm.at[sidx.at[i0 + 1]], rows_b, sem_b).wait()
                pltpu.sync_copy(rows_b, acc.at[didx.at[i0 + 1]], add=True)
                return carry

            lax.fori_loop(0, nb // 2, body, 0)
            plsc.subcore_barrier()
            pltpu.sync_copy(acc.at[pl.ds(sid * rpt, rpt)],
                            out_hbm.at[ph].at[cid].at[pl.ds(sid * rpt, rpt)])
            plsc.subcore_barrier()

        phase(ua_hbm, 0)
        phase(ub_hbm, 1)

    return scatter_kernel


def _row_spec(r=R_BLK, w=D):
    return pl.BlockSpec((r, w), lambda i: (i, 0))


def _fixed_spec(h, w):
    return pl.BlockSpec((h, w), lambda i: (0, 0))


def _tc_pre_body(x_ref, wl_ref, bl_ref, w1_ref, d0_ref, d1_ref,
                 init_ref, u1a_ref, u1b_ref, dinv_ref):
    x = x_ref[...]
    init = jnp.dot(x, wl_ref[...], preferred_element_type=jnp.float32)
    init = init + bl_ref[...]
    deg = d0_ref[..., 0] + d1_ref[..., 0] + 1.0
    dinv = lax.rsqrt(deg)[:, None]
    t1 = jnp.dot(init, w1_ref[...], preferred_element_type=jnp.float32)
    u1 = dinv * t1
    init_ref[...] = init
    u1a_ref[...] = u1[:, :DH]
    u1b_ref[...] = u1[:, DH:]
    dinv_ref[...] = dinv


def _tc_pre(xp, wl, bl, w1, d0, d1):
    return pl.pallas_call(
        _tc_pre_body,
        grid=(N_GRID,),
        in_specs=[
            _row_spec(), _fixed_spec(D, D), _fixed_spec(1, D),
            _fixed_spec(D, D), _row_spec(w=DEG_W), _row_spec(w=DEG_W),
        ],
        out_specs=[_row_spec(), _row_spec(w=DH), _row_spec(w=DH),
                   _row_spec(w=1)],
        out_shape=[
            jax.ShapeDtypeStruct((N_PAD, D), jnp.float32),
            jax.ShapeDtypeStruct((N_PAD, DH), jnp.float32),
            jax.ShapeDtypeStruct((N_PAD, DH), jnp.float32),
            jax.ShapeDtypeStruct((N_PAD, 1), jnp.float32),
        ],
    )(xp, wl, bl, w1, d0, d1)


def _acc_sum(a0a_ref, a1a_ref, a0b_ref, a1b_ref):
    return jnp.concatenate(
        [a0a_ref[...] + a1a_ref[...], a0b_ref[...] + a1b_ref[...]], axis=1)


def _u_full(ua_ref, ub_ref):
    return jnp.concatenate([ua_ref[...], ub_ref[...]], axis=1)


def _tc_mid_body(a0a_ref, a1a_ref, a0b_ref, a1b_ref, u1a_ref, u1b_ref,
                 b1_ref, w2_ref, dinv_ref, u2a_ref, u2b_ref):
    dinv = dinv_ref[...]
    m1 = dinv * (_acc_sum(a0a_ref, a1a_ref, a0b_ref, a1b_ref)
                 + _u_full(u1a_ref, u1b_ref)) + b1_ref[...]
    h1 = jnp.maximum(m1, 0.0)
    u2 = dinv * jnp.dot(h1, w2_ref[...], preferred_element_type=jnp.float32)
    u2a_ref[...] = u2[:, :DH]
    u2b_ref[...] = u2[:, DH:]


def _tc_mid(acc1, u1a, u1b, b1, w2, dinv):
    return pl.pallas_call(
        _tc_mid_body,
        grid=(N_GRID,),
        in_specs=[
            _row_spec(w=DH), _row_spec(w=DH), _row_spec(w=DH),
            _row_spec(w=DH), _row_spec(w=DH), _row_spec(w=DH),
            _fixed_spec(1, D), _fixed_spec(D, D), _row_spec(w=1),
        ],
        out_specs=[_row_spec(w=DH), _row_spec(w=DH)],
        out_shape=[
            jax.ShapeDtypeStruct((N_PAD, DH), jnp.float32),
            jax.ShapeDtypeStruct((N_PAD, DH), jnp.float32),
        ],
    )(acc1[0, 0], acc1[0, 1], acc1[1, 0], acc1[1, 1], u1a, u1b,
      b1, w2, dinv)


def _tc_post_v_body(a0a_ref, a1a_ref, a0b_ref, a1b_ref, u2a_ref, u2b_ref,
                    b2_ref, dinv_ref, init_ref, cur_ref, vc_ref):
    i = pl.program_id(0)
    h2 = dinv_ref[...] * (_acc_sum(a0a_ref, a1a_ref, a0b_ref, a1b_ref)
                          + _u_full(u2a_ref, u2b_ref)) + b2_ref[...]
    rows = lax.broadcasted_iota(jnp.int32, (R_BLK, 1), 0) + i * R_BLK
    valid = rows < N
    s = jnp.sum(jnp.where(valid, h2, 0.0), axis=0, keepdims=True)
    cursel = rows == cur_ref[0]
    rowv = jnp.sum(jnp.where(cursel, h2 + init_ref[...], 0.0),
                   axis=0, keepdims=True)

    @pl.when(i == 0)
    def _():
        vc_ref[...] = jnp.zeros_like(vc_ref)

    vc_ref[...] += (2.0 / N) * s + rowv


def _tc_post_v(acc2, u2a, u2b, b2, dinv, initp, cur):
    return pl.pallas_call(
        _tc_post_v_body,
        grid=(N_GRID,),
        in_specs=[
            _row_spec(w=DH), _row_spec(w=DH), _row_spec(w=DH),
            _row_spec(w=DH), _row_spec(w=DH), _row_spec(w=DH),
            _fixed_spec(1, D), _row_spec(w=1), _row_spec(),
            pl.BlockSpec(memory_space=pltpu.SMEM),
        ],
        out_specs=_fixed_spec(1, D),
        out_shape=jax.ShapeDtypeStruct((1, D), jnp.float32),
    )(acc2[0, 0], acc2[0, 1], acc2[1, 0], acc2[1, 1], u2a, u2b,
      b2, dinv, initp, cur)


def _tc_post_p_body(a0a_ref, a1a_ref, a0b_ref, a1b_ref, u2a_ref, u2b_ref,
                    b2_ref, dinv_ref, init_ref, h2i_ref, ps_ref):
    i = pl.program_id(0)
    h2 = dinv_ref[...] * (_acc_sum(a0a_ref, a1a_ref, a0b_ref, a1b_ref)
                          + _u_full(u2a_ref, u2b_ref)) + b2_ref[...]
    rows = lax.broadcasted_iota(jnp.int32, (R_BLK, 1), 0) + i * R_BLK
    valid = rows < N
    s = jnp.sum(jnp.where(valid, h2, 0.0), axis=0, keepdims=True)
    h2i_ref[...] = h2 + init_ref[...]

    @pl.when(i == 0)
    def _():
        ps_ref[...] = jnp.zeros_like(ps_ref)

    ps_ref[...] += (1.0 / N) * s


def _tc_post_p(acc2, u2a, u2b, b2, dinv, initp):
    return pl.pallas_call(
        _tc_post_p_body,
        grid=(N_GRID,),
        in_specs=[
            _row_spec(w=DH), _row_spec(w=DH), _row_spec(w=DH),
            _row_spec(w=DH), _row_spec(w=DH), _row_spec(w=DH),
            _fixed_spec(1, D), _row_spec(w=1), _row_spec(),
        ],
        out_specs=[_row_spec(), _fixed_spec(1, D)],
        out_shape=[
            jax.ShapeDtypeStruct((N_PAD, D), jnp.float32),
            jax.ShapeDtypeStruct((1, D), jnp.float32),
        ],
    )(acc2[0, 0], acc2[0, 1], acc2[1, 0], acc2[1, 1], u2a, u2b,
      b2, dinv, initp)


def _tc_final_body(h2i_ref, pg_ref, vc_ref, out_ref):
    out_ref[...] = h2i_ref[...] + (pg_ref[...] + vc_ref[...])


def _tc_final(h2i, pg, vc):
    return pl.pallas_call(
        _tc_final_body,
        grid=(N_GRID,),
        in_specs=[_row_spec(), _fixed_spec(1, D), _fixed_spec(1, D)],
        out_specs=_row_spec(),
        out_shape=jax.ShapeDtypeStruct((N_PAD, D), jnp.float32),
    )(h2i, pg, vc)


def kernel(v_net_x, v_net_edge_index, v_net_batch, p_net_x, p_net_edge_index,
           p_net_batch, curr_v_node_id, v_Wl, v_bl, v_W1, v_b1, v_W2, v_b2,
           p_Wl, p_bl, p_W1, p_b1, p_W2, p_b2):
    e_v = v_net_edge_index.shape[1]
    e_p = p_net_edge_index.shape[1]
    ev_pad = -(-e_v // (NW * EB)) * (NW * EB)
    ep_pad = -(-e_p // (NW * EB)) * (NW * EB)
    nb_v = ev_pad // (NW * EB)
    nb_p = ep_pad // (NW * EB)

    srcv, dstv = _pad_edges(v_net_edge_index, ev_pad)
    srcp, dstp = _pad_edges(p_net_edge_index, ep_pad)

    zeros_nd = jnp.zeros((N_PAD, DH), jnp.float32)
    zeros_nw = jnp.zeros((N_PAD, DEG_W), jnp.float32)
    ones_bw = jnp.ones((EB, DEG_W), jnp.float32)

    xv = jnp.pad(v_net_x, ((0, N_PAD - N), (0, 0)))
    xp = jnp.pad(p_net_x, ((0, N_PAD - N), (0, 0)))

    degv, degp = _make_deg_kernel(nb_v, nb_p)(dstv, dstp, ones_bw, zeros_nw)

    scat_v = _make_scatter_kernel(nb_v)
    scat_p = _make_scatter_kernel(nb_p)

    initv, u1va, u1vb, dinvv = _tc_pre(xv, v_Wl, v_bl.reshape(1, D), v_W1,
                                       degv[0], degv[1])
    acc1v = scat_v(u1va, u1vb, srcv, dstv, zeros_nd)
    u2va, u2vb = _tc_mid(acc1v, u1va, u1vb, v_b1.reshape(1, D), v_W2, dinvv)
    acc2v = scat_v(u2va, u2vb, srcv, dstv, zeros_nd)

    initp, u1pa, u1pb, dinvp = _tc_pre(xp, p_Wl, p_bl.reshape(1, D), p_W1,
                                       degp[0], degp[1])
    acc1p = scat_p(u1pa, u1pb, srcp, dstp, zeros_nd)
    u2pa, u2pb = _tc_mid(acc1p, u1pa, u1pb, p_b1.reshape(1, D), p_W2, dinvp)
    acc2p = scat_p(u2pa, u2pb, srcp, dstp, zeros_nd)

    vconst = _tc_post_v(acc2v, u2va, u2vb, v_b2.reshape(1, D), dinvv,
                        initv, curr_v_node_id.astype(jnp.int32))
    h2i, pg = _tc_post_p(acc2p, u2pa, u2pb, p_b2.reshape(1, D), dinvp, initp)
    out = _tc_final(h2i, pg, vconst)
    return out[None, :N, :]
